# Initial kernel scaffold; baseline (speedup 1.0000x reference)
import functools

import jax
import jax.numpy as jnp
from jax import lax
from jax.experimental import pallas as pl
from jax.experimental.pallas import tpu as pltpu
from jax.experimental.pallas import tpu_sc as plsc

F32 = jnp.float32
NC = 2
NS = 16
NW = NC * NS
CH = 128


def _edge_feats_body(ea, We1, be1, We2, be2, We3, be3, o1, o2, o3):
    a = ea[...]
    o1[...] = jnp.dot(a, We1[...], preferred_element_type=F32) + be1[...]
    o2[...] = jnp.dot(a, We2[...], preferred_element_type=F32) + be2[...]
    o3[...] = jnp.dot(a, We3[...], preferred_element_type=F32) + be3[...]


def _edge_feats(edge_attr, We1, be1, We2, be2, We3, be3):
    E, DE = edge_attr.shape
    D1 = We1.shape[1]
    D2 = We2.shape[1]
    D3 = We3.shape[1]
    EB = 3200
    grid = (E // EB,)
    full = lambda s: pl.BlockSpec(s, lambda i: (0, 0))
    return pl.pallas_call(
        _edge_feats_body,
        grid=grid,
        in_specs=[
            pl.BlockSpec((EB, DE), lambda i: (i, 0)),
            full((DE, D1)), full((1, D1)),
            full((DE, D2)), full((1, D2)),
            full((DE, D3)), full((1, D3)),
        ],
        out_specs=[
            pl.BlockSpec((EB, D1), lambda i: (i, 0)),
            pl.BlockSpec((EB, D2), lambda i: (i, 0)),
            pl.BlockSpec((EB, D3), lambda i: (i, 0)),
        ],
        out_shape=[
            jax.ShapeDtypeStruct((E, D1), F32),
            jax.ShapeDtypeStruct((E, D2), F32),
            jax.ShapeDtypeStruct((E, D3), F32),
        ],
    )(edge_attr, We1, be1.reshape(1, -1), We2, be2.reshape(1, -1),
      We3, be3.reshape(1, -1))


def _mlp_body(t, a0, a1, Wa, ba, g, bt, Wb, bb, o):
    s = t[...] + a0[...] + a1[...]
    h = jnp.dot(s, Wa[...], preferred_element_type=F32) + ba[...]
    n = h.shape[0]
    mu = jnp.sum(h, axis=0, keepdims=True) * (1.0 / n)
    d = h - mu
    var = jnp.sum(d * d, axis=0, keepdims=True) * (1.0 / n)
    h = d * lax.rsqrt(var + 1e-5) * g[...] + bt[...]
    h = jnp.maximum(h, 0.0)
    h = jnp.maximum(jnp.dot(h, Wb[...], preferred_element_type=F32) + bb[...], 0.0)
    o[...] = h


def _mlp(t, a0, a1, Wa, ba, g, bt, Wb, bb):
    n = t.shape[0]
    H = Wa.shape[1]
    return pl.pallas_call(
        _mlp_body,
        out_shape=jax.ShapeDtypeStruct((n, H), F32),
    )(t, a0, a1, Wa, ba.reshape(1, -1), g.reshape(1, -1), bt.reshape(1, -1),
      Wb, bb.reshape(1, -1))


def _head_body(h1, h2, h3, W1, W2, W3, bl1, Wl2, bl2, o):
    z = (jnp.dot(h1[...], W1[...], preferred_element_type=F32)
         + jnp.dot(h2[...], W2[...], preferred_element_type=F32)
         + jnp.dot(h3[...], W3[...], preferred_element_type=F32)
         + bl1[...])
    z = jnp.maximum(z, 0.0)
    o[...] = jnp.dot(z, Wl2[...], preferred_element_type=F32) + bl2[...]


def _head(h1, h2, h3, Wl1, bl1, Wl2, bl2):
    n, H = h1.shape
    DO = Wl2.shape[1]
    return pl.pallas_call(
        _head_body,
        out_shape=jax.ShapeDtypeStruct((n, DO), F32),
    )(h1, h2, h3, Wl1[:H], Wl1[H:2 * H], Wl1[2 * H:], bl1.reshape(1, -1),
      Wl2, bl2.reshape(1, -1))


@functools.lru_cache(maxsize=None)
def _make_msg_kernel(n_nodes, n_edges, Df):
    n_chunks = n_edges // CH
    rows_per_tile = n_nodes // NS
    mesh = plsc.VectorSubcoreMesh(core_axis_name="c", subcore_axis_name="s")

    @functools.partial(
        pl.kernel,
        out_type=jax.ShapeDtypeStruct((NC, n_nodes, Df), F32),
        mesh=mesh,
        scratch_types=[
            pltpu.VMEM((CH,), jnp.int32),
            pltpu.VMEM((CH,), jnp.int32),
            pltpu.VMEM((CH, Df), F32),
            pltpu.VMEM((CH, Df), F32),
            pltpu.VMEM_SHARED((n_nodes, Df), F32),
            pltpu.SemaphoreType.DMA,
        ],
    )
    def msg(table, src, dst, ef, zeros, out, sidx, didx, xbuf, ebuf, aggr, sem):
        c = lax.axis_index("c")
        s = lax.axis_index("s")
        wid = s * NC + c

        stripe = pl.ds(s * rows_per_tile, rows_per_tile)
        pltpu.sync_copy(zeros.at[stripe], aggr.at[stripe])
        plsc.subcore_barrier()

        n_mine = (n_chunks - wid + NW - 1) // NW

        def chunk_body(i, carry):
            base = (wid + i * NW) * CH
            pltpu.sync_copy(src.at[pl.ds(base, CH)], sidx)
            pltpu.sync_copy(ef.at[pl.ds(base, CH)], ebuf)
            pltpu.async_copy(table.at[sidx], xbuf, sem).wait()
            pltpu.sync_copy(dst.at[pl.ds(base, CH)], didx)

            def row_body(r, carry2):
                for j in range(Df // 16):
                    sl = pl.ds(j * 16, 16)
                    ebuf[r, sl] = jnp.maximum(xbuf[r, sl] + ebuf[r, sl], 0.0)
                return carry2

            lax.fori_loop(0, CH, row_body, 0)
            pltpu.sync_copy(ebuf, aggr.at[didx], add=True)
            return carry

        lax.fori_loop(0, n_mine, chunk_body, 0)
        plsc.subcore_barrier()
        pltpu.sync_copy(aggr.at[stripe], out.at[c, stripe])

    return msg


def _message(table, src, dst, ef, zeros):
    n_nodes, Df = table.shape
    k = _make_msg_kernel(n_nodes, src.shape[0], Df)
    parts = k(table, src, dst, ef, zeros)
    return parts[0], parts[1]


def kernel(x, edge_index, edge_attr, batch,
           We1, be1, Wa1, ba1, g1, bt1, Wb1, bb1,
           We2, be2, Wa2, ba2, g2, bt2, Wb2, bb2,
           We3, be3, Wa3, ba3, g3, bt3, Wb3, bb3,
           Wl1, bl1, Wl2, bl2):
    src = edge_index[0]
    dst = edge_index[1]
    n = x.shape[0]

    e1, e2, e3 = _edge_feats(edge_attr, We1, be1, We2, be2, We3, be3)

    z128 = jnp.zeros((n, x.shape[1]), F32)
    z16 = jnp.zeros((n, Wa2.shape[0]), F32)

    a0, a1 = _message(x, src, dst, e1, z128)
    h1 = _mlp(x, a0, a1, Wa1, ba1, g1, bt1, Wb1, bb1)

    a0, a1 = _message(h1, src, dst, e2, z16)
    h2 = _mlp(h1, a0, a1, Wa2, ba2, g2, bt2, Wb2, bb2)

    a0, a1 = _message(h2, src, dst, e3, z16)
    h3 = _mlp(h2, a0, a1, Wa3, ba3, g3, bt3, Wb3, bb3)

    return _head(h1, h2, h3, Wl1, bl1, Wl2, bl2)


# speedup vs baseline: 2.9269x; 2.9269x over previous
import functools

import jax
import jax.numpy as jnp
from jax import lax
from jax.experimental import pallas as pl
from jax.experimental.pallas import tpu as pltpu
from jax.experimental.pallas import tpu_sc as plsc

F32 = jnp.float32
NC = 2
NS = 16
NW = NC * NS
CH = 128


def _edge_feats_body(ea, We1, be1, We2, be2, We3, be3, o1, o2, o3):
    a = ea[...]
    o1[...] = jnp.dot(a, We1[...], preferred_element_type=F32) + be1[...]
    o2[...] = jnp.dot(a, We2[...], preferred_element_type=F32) + be2[...]
    o3[...] = jnp.dot(a, We3[...], preferred_element_type=F32) + be3[...]


def _edge_feats(edge_attr, We1, be1, We2, be2, We3, be3):
    E, DE = edge_attr.shape
    D1 = We1.shape[1]
    D2 = We2.shape[1]
    D3 = We3.shape[1]
    EB = 3200
    grid = (E // EB,)
    full = lambda s: pl.BlockSpec(s, lambda i: (0, 0))
    return pl.pallas_call(
        _edge_feats_body,
        grid=grid,
        in_specs=[
            pl.BlockSpec((EB, DE), lambda i: (i, 0)),
            full((DE, D1)), full((1, D1)),
            full((DE, D2)), full((1, D2)),
            full((DE, D3)), full((1, D3)),
        ],
        out_specs=[
            pl.BlockSpec((EB, D1), lambda i: (i, 0)),
            pl.BlockSpec((EB, D2), lambda i: (i, 0)),
            pl.BlockSpec((EB, D3), lambda i: (i, 0)),
        ],
        out_shape=[
            jax.ShapeDtypeStruct((E, D1), F32),
            jax.ShapeDtypeStruct((E, D2), F32),
            jax.ShapeDtypeStruct((E, D3), F32),
        ],
    )(edge_attr, We1, be1.reshape(1, -1), We2, be2.reshape(1, -1),
      We3, be3.reshape(1, -1))


def _mlp_body(t, a0, a1, Wa, ba, g, bt, Wb, bb, P, o):
    Din = Wa.shape[0]
    s = (t[...] + a0[...] + a1[...])[:, :Din]
    h = jnp.dot(s, Wa[...], preferred_element_type=F32) + ba[...]
    n = h.shape[0]
    mu = jnp.sum(h, axis=0, keepdims=True) * (1.0 / n)
    d = h - mu
    var = jnp.sum(d * d, axis=0, keepdims=True) * (1.0 / n)
    h = d * lax.rsqrt(var + 1e-5) * g[...] + bt[...]
    h = jnp.maximum(h, 0.0)
    h = jnp.maximum(jnp.dot(h, Wb[...], preferred_element_type=F32) + bb[...], 0.0)
    o[...] = jnp.dot(h, P[...], preferred_element_type=F32)


def _mlp(t, a0, a1, Wa, ba, g, bt, Wb, bb, P):
    n = t.shape[0]
    return pl.pallas_call(
        _mlp_body,
        out_shape=jax.ShapeDtypeStruct((n, P.shape[1]), F32),
    )(t, a0, a1, Wa, ba.reshape(1, -1), g.reshape(1, -1), bt.reshape(1, -1),
      Wb, bb.reshape(1, -1), P)


def _head_body(h1, h2, h3, W1, W2, W3, bl1, Wl2, bl2, o):
    H = W1.shape[0]
    z = (jnp.dot(h1[...][:, :H], W1[...], preferred_element_type=F32)
         + jnp.dot(h2[...][:, :H], W2[...], preferred_element_type=F32)
         + jnp.dot(h3[...][:, :H], W3[...], preferred_element_type=F32)
         + bl1[...])
    z = jnp.maximum(z, 0.0)
    o[...] = jnp.dot(z, Wl2[...], preferred_element_type=F32) + bl2[...]


def _head(h1, h2, h3, Wl1, bl1, Wl2, bl2):
    n = h1.shape[0]
    H = Wl1.shape[0] // 3
    DO = Wl2.shape[1]
    return pl.pallas_call(
        _head_body,
        out_shape=jax.ShapeDtypeStruct((n, DO), F32),
    )(h1, h2, h3, Wl1[:H], Wl1[H:2 * H], Wl1[2 * H:], bl1.reshape(1, -1),
      Wl2, bl2.reshape(1, -1))


@functools.lru_cache(maxsize=None)
def _make_msg_kernel(n_pad, n_edges, De):
    Df = 128
    n_chunks = n_edges // CH
    rows_per_tile = n_pad // NS
    mesh = plsc.VectorSubcoreMesh(core_axis_name="c", subcore_axis_name="s")

    scratch = [
        pltpu.VMEM((CH,), jnp.int32),
        pltpu.VMEM((CH,), jnp.int32),
        pltpu.VMEM((CH, Df), F32),
        pltpu.VMEM((CH, Df), F32),
        pltpu.VMEM_SHARED((n_pad, Df), F32),
        pltpu.SemaphoreType.DMA,
    ]
    if De < Df:
        scratch.append(pltpu.VMEM((CH, De), F32))

    @functools.partial(
        pl.kernel,
        out_type=jax.ShapeDtypeStruct((NC, n_pad, Df), F32),
        mesh=mesh,
        scratch_types=scratch,
    )
    def msg(table, src, dst, ef, zeros, out, sidx, didx, xbuf, ebuf, aggr, sem,
            *rest):
        c = lax.axis_index("c")
        s = lax.axis_index("s")
        wid = s * NC + c

        stripe = pl.ds(s * rows_per_tile, rows_per_tile)
        pltpu.sync_copy(zeros.at[stripe], aggr.at[stripe])
        if De < Df:
            efbuf = rest[0]
            pltpu.sync_copy(zeros.at[pl.ds(0, CH)], ebuf)
        plsc.subcore_barrier()

        n_mine = (n_chunks - wid + NW - 1) // NW

        def chunk_body(i, carry):
            base = (wid + i * NW) * CH
            pltpu.sync_copy(src.at[pl.ds(base, CH)], sidx)
            if De < Df:
                pltpu.sync_copy(ef.at[pl.ds(base, CH)], efbuf)
            else:
                pltpu.sync_copy(ef.at[pl.ds(base, CH)], ebuf)
            pltpu.async_copy(table.at[sidx], xbuf, sem).wait()
            pltpu.sync_copy(dst.at[pl.ds(base, CH)], didx)

            def row_body(r, carry2):
                for j in range(De // 16):
                    sl = pl.ds(j * 16, 16)
                    e = efbuf[r, sl] if De < Df else ebuf[r, sl]
                    ebuf[r, sl] = jnp.maximum(xbuf[r, sl] + e, 0.0)
                return carry2

            lax.fori_loop(0, CH, row_body, 0)
            pltpu.sync_copy(ebuf, aggr.at[didx], add=True)
            return carry

        lax.fori_loop(0, n_mine, chunk_body, 0)
        plsc.subcore_barrier()
        pltpu.sync_copy(aggr.at[stripe], out.at[c, stripe])

    return msg


def _message(table, src, dst, ef, zeros):
    k = _make_msg_kernel(zeros.shape[0], src.shape[0], ef.shape[1])
    parts = k(table, src, dst, ef, zeros)
    n_nodes = table.shape[0]
    return parts[0, :n_nodes], parts[1, :n_nodes]


def kernel(x, edge_index, edge_attr, batch,
           We1, be1, Wa1, ba1, g1, bt1, Wb1, bb1,
           We2, be2, Wa2, ba2, g2, bt2, Wb2, bb2,
           We3, be3, Wa3, ba3, g3, bt3, Wb3, bb3,
           Wl1, bl1, Wl2, bl2):
    src = edge_index[0]
    dst = edge_index[1]
    n = x.shape[0]

    e1, e2, e3 = _edge_feats(edge_attr, We1, be1, We2, be2, We3, be3)

    D = x.shape[1]
    H = Wa2.shape[0]
    n_pad = ((n + 8 * NS - 1) // (8 * NS)) * (8 * NS)
    z128 = jnp.zeros((n_pad, D), F32)
    P = jnp.concatenate([jnp.eye(H, dtype=F32),
                         jnp.zeros((H, D - H), F32)], axis=1)

    a0, a1 = _message(x, src, dst, e1, z128)
    h1 = _mlp(x, a0, a1, Wa1, ba1, g1, bt1, Wb1, bb1, P)

    a0, a1 = _message(h1, src, dst, e2, z128)
    h2 = _mlp(h1, a0, a1, Wa2, ba2, g2, bt2, Wb2, bb2, P)

    a0, a1 = _message(h2, src, dst, e3, z128)
    h3 = _mlp(h2, a0, a1, Wa3, ba3, g3, bt3, Wb3, bb3, P)

    return _head(h1, h2, h3, Wl1, bl1, Wl2, bl2)


# speedup vs baseline: 3.6526x; 1.2479x over previous
import functools

import jax
import jax.numpy as jnp
from jax import lax
from jax.experimental import pallas as pl
from jax.experimental.pallas import tpu as pltpu
from jax.experimental.pallas import tpu_sc as plsc

F32 = jnp.float32
NC = 2
NS = 16
NW = NC * NS
CH = 128


def _edge_feats_body(ea, We1, be1, We2, be2, We3, be3, o1, o2, o3):
    a = ea[...]
    o1[...] = jnp.dot(a, We1[...], preferred_element_type=F32) + be1[...]
    o2[...] = jnp.dot(a, We2[...], preferred_element_type=F32) + be2[...]
    o3[...] = jnp.dot(a, We3[...], preferred_element_type=F32) + be3[...]


def _edge_feats(edge_attr, We1, be1, We2, be2, We3, be3):
    E, DE = edge_attr.shape
    D1 = We1.shape[1]
    D2 = We2.shape[1]
    D3 = We3.shape[1]
    EB = 4096
    grid = (E // EB,)
    full = lambda s: pl.BlockSpec(s, lambda i: (0, 0))
    return pl.pallas_call(
        _edge_feats_body,
        grid=grid,
        in_specs=[
            pl.BlockSpec((EB, DE), lambda i: (i, 0)),
            full((DE, D1)), full((1, D1)),
            full((DE, D2)), full((1, D2)),
            full((DE, D3)), full((1, D3)),
        ],
        out_specs=[
            pl.BlockSpec((EB, D1), lambda i: (i, 0)),
            pl.BlockSpec((EB, D2), lambda i: (i, 0)),
            pl.BlockSpec((EB, D3), lambda i: (i, 0)),
        ],
        out_shape=[
            jax.ShapeDtypeStruct((E, D1), F32),
            jax.ShapeDtypeStruct((E, D2), F32),
            jax.ShapeDtypeStruct((E, D3), F32),
        ],
    )(edge_attr, We1, be1.reshape(1, -1), We2, be2.reshape(1, -1),
      We3, be3.reshape(1, -1))


def _mlp_body(t, a0, a1, Wa, ba, g, bt, Wb, bb, P, o):
    Din = Wa.shape[0]
    s = (t[...] + a0[...] + a1[...])[:, :Din]
    h = jnp.dot(s, Wa[...], preferred_element_type=F32) + ba[...]
    n = h.shape[0]
    mu = jnp.sum(h, axis=0, keepdims=True) * (1.0 / n)
    d = h - mu
    var = jnp.sum(d * d, axis=0, keepdims=True) * (1.0 / n)
    h = d * lax.rsqrt(var + 1e-5) * g[...] + bt[...]
    h = jnp.maximum(h, 0.0)
    h = jnp.maximum(jnp.dot(h, Wb[...], preferred_element_type=F32) + bb[...], 0.0)
    o[...] = jnp.dot(h, P[...], preferred_element_type=F32)


def _mlp(t, a0, a1, Wa, ba, g, bt, Wb, bb, P):
    n = t.shape[0]
    return pl.pallas_call(
        _mlp_body,
        out_shape=jax.ShapeDtypeStruct((n, P.shape[1]), F32),
    )(t, a0, a1, Wa, ba.reshape(1, -1), g.reshape(1, -1), bt.reshape(1, -1),
      Wb, bb.reshape(1, -1), P)


def _head_body(h1, h2, h3, W1, W2, W3, bl1, Wl2, bl2, o):
    H = W1.shape[0]
    z = (jnp.dot(h1[...][:, :H], W1[...], preferred_element_type=F32)
         + jnp.dot(h2[...][:, :H], W2[...], preferred_element_type=F32)
         + jnp.dot(h3[...][:, :H], W3[...], preferred_element_type=F32)
         + bl1[...])
    z = jnp.maximum(z, 0.0)
    o[...] = jnp.dot(z, Wl2[...], preferred_element_type=F32) + bl2[...]


def _head(h1, h2, h3, Wl1, bl1, Wl2, bl2):
    n = h1.shape[0]
    H = Wl1.shape[0] // 3
    DO = Wl2.shape[1]
    return pl.pallas_call(
        _head_body,
        out_shape=jax.ShapeDtypeStruct((n, DO), F32),
    )(h1, h2, h3, Wl1[:H], Wl1[H:2 * H], Wl1[2 * H:], bl1.reshape(1, -1),
      Wl2, bl2.reshape(1, -1))


@functools.lru_cache(maxsize=None)
def _make_msg_kernel(n_pad, n_edges, De, ch):
    Df = 128
    n_chunks = n_edges // ch
    KPT = n_chunks // NW
    rows_per_tile = n_pad // NS
    mesh = plsc.VectorSubcoreMesh(core_axis_name="c", subcore_axis_name="s")

    SB = 16
    scratch = [
        pltpu.VMEM((SB, ch), jnp.int32),
        pltpu.VMEM((SB, ch), jnp.int32),
        pltpu.VMEM((2, ch, Df), F32),
        pltpu.VMEM((2, ch, De), F32),
        pltpu.VMEM_SHARED((n_pad, Df), F32),
        pltpu.SemaphoreType.DMA, pltpu.SemaphoreType.DMA,
        pltpu.SemaphoreType.DMA, pltpu.SemaphoreType.DMA,
        pltpu.SemaphoreType.DMA, pltpu.SemaphoreType.DMA,
    ]

    @functools.partial(
        pl.kernel,
        out_type=jax.ShapeDtypeStruct((NC, n_pad, Df), F32),
        mesh=mesh,
        scratch_types=scratch,
    )
    def msg(table, src2d, dst2d, ef, zeros, out, sidx, didx, xbuf, efbuf,
            aggr, sem_g0, sem_g1, sem_e0, sem_e1, sem_s0, sem_s1):
        c = lax.axis_index("c")
        s = lax.axis_index("s")
        wid = s * NC + c
        k0 = wid * KPT
        sem_g = (sem_g0, sem_g1)
        sem_e = (sem_e0, sem_e1)
        sem_s = (sem_s0, sem_s1)

        stripe = pl.ds(s * rows_per_tile, rows_per_tile)
        pltpu.sync_copy(zeros, aggr.at[stripe])
        plsc.subcore_barrier()

        def compute(b):
            def row_body(r, carry2):
                for j in range(De // 16):
                    sl = pl.ds(j * 16, 16)
                    xbuf[b, r, sl] = jnp.maximum(
                        xbuf[b, r, sl] + efbuf[b, r, sl], 0.0)
                return carry2

            lax.fori_loop(0, ch, row_body, 0)

        @pl.loop(0, KPT, step=2)
        def pair_body(k2):
            @pl.when(lax.rem(k2, SB) == 0)
            def _():
                kb = pl.multiple_of(k0 + k2, SB)
                pltpu.sync_copy(src2d.at[pl.ds(kb, SB)], sidx)
                pltpu.sync_copy(dst2d.at[pl.ds(kb, SB)], didx)

            r = lax.rem(k2, SB)
            g0 = pltpu.async_copy(table.at[sidx.at[r]], xbuf.at[0], sem_g[0])
            f0 = pltpu.async_copy(ef.at[pl.ds((k0 + k2) * ch, ch)],
                                  efbuf.at[0], sem_e[0])
            g1 = pltpu.async_copy(table.at[sidx.at[r + 1]], xbuf.at[1],
                                  sem_g[1])
            f1 = pltpu.async_copy(ef.at[pl.ds((k0 + k2 + 1) * ch, ch)],
                                  efbuf.at[1], sem_e[1])
            g0.wait()
            f0.wait()
            compute(0)
            s0 = pltpu.async_copy(xbuf.at[0], aggr.at[didx.at[r]], sem_s[0],
                                  add=True)
            g1.wait()
            f1.wait()
            compute(1)
            s1 = pltpu.async_copy(xbuf.at[1], aggr.at[didx.at[r + 1]],
                                  sem_s[1], add=True)
            s0.wait()
            s1.wait()

        plsc.subcore_barrier()
        pltpu.sync_copy(aggr.at[stripe], out.at[c, stripe])

    return msg


def _message(table, src2d, dst2d, ef, zeros, n_pad):
    ch = src2d.shape[1]
    k = _make_msg_kernel(n_pad, ef.shape[0], ef.shape[1], ch)
    parts = k(table, src2d, dst2d, ef, zeros)
    n_nodes = table.shape[0]
    return parts[0, :n_nodes], parts[1, :n_nodes]


def kernel(x, edge_index, edge_attr, batch,
           We1, be1, Wa1, ba1, g1, bt1, Wb1, bb1,
           We2, be2, Wa2, ba2, g2, bt2, Wb2, bb2,
           We3, be3, Wa3, ba3, g3, bt3, Wb3, bb3,
           Wl1, bl1, Wl2, bl2):
    src = edge_index[0]
    dst = edge_index[1]
    n = x.shape[0]
    D = x.shape[1]
    H = Wa2.shape[0]
    n_pad = ((n + 8 * NS - 1) // (8 * NS)) * (8 * NS)

    E = src.shape[0]

    def pad_idx(ch):
        group = NW * ch * 16
        e_pad = -(-E // group) * group
        pad_n = e_pad - E
        s2, d2 = src, dst
        if pad_n:
            idx = jnp.arange(pad_n, dtype=jnp.int32)
            s2 = jnp.concatenate([s2, idx % n])
            d2 = jnp.concatenate([d2, n + idx % (n_pad - n)])
        return s2.reshape(-1, ch), d2.reshape(-1, ch), e_pad

    src1, dst1, E1 = pad_idx(CH // 2)
    src23, dst23, E23 = src1, dst1, E1
    edge_attr = jnp.pad(edge_attr, ((0, E1 - E), (0, 0)))

    e1, e2, e3 = _edge_feats(edge_attr, We1, be1, We2, be2, We3, be3)

    z128 = jnp.zeros((n_pad // NS, D), F32)
    P = jnp.concatenate([jnp.eye(H, dtype=F32),
                         jnp.zeros((H, D - H), F32)], axis=1)

    a0, a1 = _message(x, src1, dst1, e1[:E1], z128, n_pad)
    h1 = _mlp(x, a0, a1, Wa1, ba1, g1, bt1, Wb1, bb1, P)

    a0, a1 = _message(h1, src23, dst23, e2[:E23], z128, n_pad)
    h2 = _mlp(h1, a0, a1, Wa2, ba2, g2, bt2, Wb2, bb2, P)

    a0, a1 = _message(h2, src23, dst23, e3[:E23], z128, n_pad)
    h3 = _mlp(h2, a0, a1, Wa3, ba3, g3, bt3, Wb3, bb3, P)

    return _head(h1, h2, h3, Wl1, bl1, Wl2, bl2)


# speedup vs baseline: 3.7921x; 1.0382x over previous
import functools

import jax
import jax.numpy as jnp
from jax import lax
from jax.experimental import pallas as pl
from jax.experimental.pallas import tpu as pltpu
from jax.experimental.pallas import tpu_sc as plsc

F32 = jnp.float32
NC = 2
NS = 16
NW = NC * NS
CH = 128


def _edge_feats_body(ea, We1, be1, We2, be2, We3, be3, o1, o2, o3):
    a = ea[...]
    o1[...] = jnp.dot(a, We1[...], preferred_element_type=F32) + be1[...]
    o2[...] = jnp.dot(a, We2[...], preferred_element_type=F32) + be2[...]
    o3[...] = jnp.dot(a, We3[...], preferred_element_type=F32) + be3[...]


def _edge_feats(edge_attr, We1, be1, We2, be2, We3, be3):
    E, DE = edge_attr.shape
    D1 = We1.shape[1]
    D2 = We2.shape[1]
    D3 = We3.shape[1]
    EB = 4096
    grid = (E // EB,)
    full = lambda s: pl.BlockSpec(s, lambda i: (0, 0))
    return pl.pallas_call(
        _edge_feats_body,
        grid=grid,
        in_specs=[
            pl.BlockSpec((EB, DE), lambda i: (i, 0)),
            full((DE, D1)), full((1, D1)),
            full((DE, D2)), full((1, D2)),
            full((DE, D3)), full((1, D3)),
        ],
        out_specs=[
            pl.BlockSpec((EB, D1), lambda i: (i, 0)),
            pl.BlockSpec((EB, D2), lambda i: (i, 0)),
            pl.BlockSpec((EB, D3), lambda i: (i, 0)),
        ],
        out_shape=[
            jax.ShapeDtypeStruct((E, D1), F32),
            jax.ShapeDtypeStruct((E, D2), F32),
            jax.ShapeDtypeStruct((E, D3), F32),
        ],
    )(edge_attr, We1, be1.reshape(1, -1), We2, be2.reshape(1, -1),
      We3, be3.reshape(1, -1))


def _mlp_body(t, a0, a1, Wa, ba, g, bt, Wb, bb, P, o):
    Din = Wa.shape[0]
    s = (t[...] + a0[...] + a1[...])[:, :Din]
    h = jnp.dot(s, Wa[...], preferred_element_type=F32) + ba[...]
    n = h.shape[0]
    mu = jnp.sum(h, axis=0, keepdims=True) * (1.0 / n)
    d = h - mu
    var = jnp.sum(d * d, axis=0, keepdims=True) * (1.0 / n)
    h = d * lax.rsqrt(var + 1e-5) * g[...] + bt[...]
    h = jnp.maximum(h, 0.0)
    h = jnp.maximum(jnp.dot(h, Wb[...], preferred_element_type=F32) + bb[...], 0.0)
    o[...] = jnp.dot(h, P[...], preferred_element_type=F32)


def _mlp(t, a0, a1, Wa, ba, g, bt, Wb, bb, P):
    n = t.shape[0]
    return pl.pallas_call(
        _mlp_body,
        out_shape=jax.ShapeDtypeStruct((n, P.shape[1]), F32),
    )(t, a0, a1, Wa, ba.reshape(1, -1), g.reshape(1, -1), bt.reshape(1, -1),
      Wb, bb.reshape(1, -1), P)


def _head_body(h1, h2, h3, W1, W2, W3, bl1, Wl2, bl2, o):
    H = W1.shape[0]
    z = (jnp.dot(h1[...][:, :H], W1[...], preferred_element_type=F32)
         + jnp.dot(h2[...][:, :H], W2[...], preferred_element_type=F32)
         + jnp.dot(h3[...][:, :H], W3[...], preferred_element_type=F32)
         + bl1[...])
    z = jnp.maximum(z, 0.0)
    o[...] = jnp.dot(z, Wl2[...], preferred_element_type=F32) + bl2[...]


def _head(h1, h2, h3, Wl1, bl1, Wl2, bl2):
    n = h1.shape[0]
    H = Wl1.shape[0] // 3
    DO = Wl2.shape[1]
    return pl.pallas_call(
        _head_body,
        out_shape=jax.ShapeDtypeStruct((n, DO), F32),
    )(h1, h2, h3, Wl1[:H], Wl1[H:2 * H], Wl1[2 * H:], bl1.reshape(1, -1),
      Wl2, bl2.reshape(1, -1))


@functools.lru_cache(maxsize=None)
def _make_msg_kernel(n_pad, n_edges, De, ch):
    Df = 128
    n_chunks = n_edges // ch
    KPT = n_chunks // NW
    rows_per_tile = n_pad // NS
    mesh = plsc.VectorSubcoreMesh(core_axis_name="c", subcore_axis_name="s")

    SB = 16
    scratch = [
        pltpu.VMEM((SB, ch), jnp.int32),
        pltpu.VMEM((SB, ch), jnp.int32),
        pltpu.VMEM((2, ch, Df), F32),
        pltpu.VMEM((2, ch, De), F32),
        pltpu.VMEM_SHARED((n_pad, Df), F32),
        pltpu.SemaphoreType.DMA, pltpu.SemaphoreType.DMA,
        pltpu.SemaphoreType.DMA, pltpu.SemaphoreType.DMA,
        pltpu.SemaphoreType.DMA, pltpu.SemaphoreType.DMA,
    ]

    @functools.partial(
        pl.kernel,
        out_type=jax.ShapeDtypeStruct((NC, n_pad, Df), F32),
        mesh=mesh,
        scratch_types=scratch,
    )
    def msg(table, src2d, dst2d, ef, zeros, out, sidx, didx, xbuf, efbuf,
            aggr, sem_g0, sem_g1, sem_e0, sem_e1, sem_s0, sem_s1):
        c = lax.axis_index("c")
        s = lax.axis_index("s")
        wid = s * NC + c
        k0 = wid * KPT
        sem_g = (sem_g0, sem_g1)
        sem_e = (sem_e0, sem_e1)
        sem_s = (sem_s0, sem_s1)

        stripe = pl.ds(s * rows_per_tile, rows_per_tile)
        pltpu.sync_copy(zeros, aggr.at[stripe])
        plsc.subcore_barrier()

        if De == Df:
            def compute(b):
                def row_body(r, carry2):
                    xbuf[b, r] = jnp.maximum(xbuf[b, r] + efbuf[b, r], 0.0)
                    return carry2

                lax.fori_loop(0, ch, row_body, 0)
        else:
            def compute(b):
                sl = pl.ds(0, De)
                xbuf[b, :, sl] = jnp.maximum(xbuf[b, :, sl] + efbuf[b], 0.0)

        @pl.loop(0, KPT, step=2)
        def pair_body(k2):
            @pl.when(lax.rem(k2, SB) == 0)
            def _():
                kb = pl.multiple_of(k0 + k2, SB)
                pltpu.sync_copy(src2d.at[pl.ds(kb, SB)], sidx)
                pltpu.sync_copy(dst2d.at[pl.ds(kb, SB)], didx)

            r = lax.rem(k2, SB)
            g0 = pltpu.async_copy(table.at[sidx.at[r]], xbuf.at[0], sem_g[0])
            f0 = pltpu.async_copy(ef.at[pl.ds((k0 + k2) * ch, ch)],
                                  efbuf.at[0], sem_e[0])
            g1 = pltpu.async_copy(table.at[sidx.at[r + 1]], xbuf.at[1],
                                  sem_g[1])
            f1 = pltpu.async_copy(ef.at[pl.ds((k0 + k2 + 1) * ch, ch)],
                                  efbuf.at[1], sem_e[1])
            g0.wait()
            f0.wait()
            compute(0)
            s0 = pltpu.async_copy(xbuf.at[0], aggr.at[didx.at[r]], sem_s[0],
                                  add=True)
            g1.wait()
            f1.wait()
            compute(1)
            s1 = pltpu.async_copy(xbuf.at[1], aggr.at[didx.at[r + 1]],
                                  sem_s[1], add=True)
            s0.wait()
            s1.wait()

        plsc.subcore_barrier()
        pltpu.sync_copy(aggr.at[stripe], out.at[c, stripe])

    return msg


def _message(table, src2d, dst2d, ef, zeros, n_pad):
    ch = src2d.shape[1]
    k = _make_msg_kernel(n_pad, ef.shape[0], ef.shape[1], ch)
    parts = k(table, src2d, dst2d, ef, zeros)
    n_nodes = table.shape[0]
    return parts[0, :n_nodes], parts[1, :n_nodes]


def kernel(x, edge_index, edge_attr, batch,
           We1, be1, Wa1, ba1, g1, bt1, Wb1, bb1,
           We2, be2, Wa2, ba2, g2, bt2, Wb2, bb2,
           We3, be3, Wa3, ba3, g3, bt3, Wb3, bb3,
           Wl1, bl1, Wl2, bl2):
    src = edge_index[0]
    dst = edge_index[1]
    n = x.shape[0]
    D = x.shape[1]
    H = Wa2.shape[0]
    n_pad = ((n + 8 * NS - 1) // (8 * NS)) * (8 * NS)

    E = src.shape[0]

    def pad_idx(ch):
        group = NW * ch * 16
        e_pad = -(-E // group) * group
        pad_n = e_pad - E
        s2, d2 = src, dst
        if pad_n:
            idx = jnp.arange(pad_n, dtype=jnp.int32)
            s2 = jnp.concatenate([s2, idx % n])
            d2 = jnp.concatenate([d2, n + idx % (n_pad - n)])
        return s2.reshape(-1, ch), d2.reshape(-1, ch), e_pad

    src1, dst1, E1 = pad_idx(CH // 2)
    src23, dst23, E23 = src1, dst1, E1
    edge_attr = jnp.pad(edge_attr, ((0, E1 - E), (0, 0)))

    e1, e2, e3 = _edge_feats(edge_attr, We1, be1, We2, be2, We3, be3)

    z128 = jnp.zeros((n_pad // NS, D), F32)
    P = jnp.concatenate([jnp.eye(H, dtype=F32),
                         jnp.zeros((H, D - H), F32)], axis=1)

    a0, a1 = _message(x, src1, dst1, e1[:E1], z128, n_pad)
    h1 = _mlp(x, a0, a1, Wa1, ba1, g1, bt1, Wb1, bb1, P)

    a0, a1 = _message(h1, src23, dst23, e2[:E23], z128, n_pad)
    h2 = _mlp(h1, a0, a1, Wa2, ba2, g2, bt2, Wb2, bb2, P)

    a0, a1 = _message(h2, src23, dst23, e3[:E23], z128, n_pad)
    h3 = _mlp(h2, a0, a1, Wa3, ba3, g3, bt3, Wb3, bb3, P)

    return _head(h1, h2, h3, Wl1, bl1, Wl2, bl2)
